# Initial kernel scaffold; baseline (speedup 1.0000x reference)
import jax
import jax.numpy as jnp
from jax.experimental import pallas as pl


def kernel(x, edge_attr, params, edge_index):
    raise NotImplementedError("write your pallas kernel here")


# speedup vs baseline: 1.7834x; 1.7834x over previous
import functools

import jax
import jax.numpy as jnp
from jax import lax
from jax.experimental import pallas as pl
from jax.experimental.pallas import tpu as pltpu
from jax.experimental.pallas import tpu_sc as plsc

N_NODES = 10000
N_PAD = 10240
N_EDGES = 320000
HID = 128
OUT_DIM = 4
D1 = 128

NC, NS = 2, 16
NW = NC * NS
EDGES_PER_TILE = N_EDGES // NW
CHUNK = 80
NCHUNK = EDGES_PER_TILE // CHUNK

SEG_CHUNK = 512
NSEG = N_EDGES // SEG_CHUNK
RPT = N_PAD // NW
CBUF = SEG_CHUNK + 16

ROW_BLK = 1024
NB = N_PAD // ROW_BLK


def _make_sc_msg(D):
  mesh = plsc.VectorSubcoreMesh(
      core_axis_name="c", subcore_axis_name="s", num_cores=NC, num_subcores=NS)

  @functools.partial(
      pl.kernel,
      out_type=jax.ShapeDtypeStruct((N_EDGES, D), jnp.float32),
      mesh=mesh,
      scratch_types=[
          pltpu.VMEM((CHUNK, D), jnp.float32),
          pltpu.VMEM((CHUNK, D), jnp.float32),
          pltpu.VMEM((CHUNK,), jnp.int32),
          pltpu.SemaphoreType.DMA,
      ],
  )
  def sc_msg(h_hbm, e_hbm, src_hbm, out_hbm, msg_v, e_v, src_v, sem):
    cid = lax.axis_index("c")
    sid = lax.axis_index("s")
    wid = cid * NS + sid
    base = wid * EDGES_PER_TILE

    def chunk_body(i, _):
      off = base + i * CHUNK
      pltpu.sync_copy(src_hbm.at[pl.ds(off, CHUNK)], src_v)
      pltpu.sync_copy(e_hbm.at[pl.ds(off, CHUNK), :], e_v)
      pltpu.async_copy(h_hbm.at[src_v], msg_v, sem).wait()

      def row_body(r, _):
        for c in range(D // 16):
          sl = pl.ds(c * 16, 16)
          msg_v[r, sl] = jnp.maximum(msg_v[r, sl] + e_v[r, sl], 0.0)
        return 0

      lax.fori_loop(0, CHUNK, row_body, 0)
      pltpu.sync_copy(msg_v, out_hbm.at[pl.ds(off, CHUNK), :])
      return 0

    lax.fori_loop(0, NCHUNK, chunk_body, 0)

  return sc_msg


ZC = 128
ROWS_PER_TILE = N_PAD // NS
NZ = ROWS_PER_TILE // ZC
SEGC = 80
NSEGC = N_EDGES // NW // SEGC


def _make_sc_seg(D):
  mesh = plsc.VectorSubcoreMesh(
      core_axis_name="c", subcore_axis_name="s", num_cores=NC, num_subcores=NS)

  @functools.partial(
      pl.kernel,
      out_type=jax.ShapeDtypeStruct((NC, N_PAD, D), jnp.float32),
      mesh=mesh,
      scratch_types=[
          pltpu.VMEM((SEGC, D), jnp.float32),
          pltpu.VMEM((SEGC,), jnp.int32),
          pltpu.VMEM((ZC, D), jnp.float32),
          pltpu.VMEM_SHARED((N_PAD, D), jnp.float32),
          pltpu.SemaphoreType.DMA,
      ],
  )
  def sc_seg(msg_hbm, dst_hbm, out_hbm, msg_v, dst_v, z_v, aggr_s, sem):
    cid = lax.axis_index("c")
    sid = lax.axis_index("s")
    wid = cid * NS + sid

    zero = jnp.zeros((16,), jnp.float32)

    def zrow(r, _):
      for c in range(D // 16):
        z_v[r, pl.ds(c * 16, 16)] = zero
      return 0

    lax.fori_loop(0, ZC, zrow, 0)
    row0 = sid * ROWS_PER_TILE
    for z in range(NZ):
      pltpu.sync_copy(z_v, aggr_s.at[pl.ds(row0 + z * ZC, ZC), :])
    plsc.subcore_barrier()

    base = wid * (N_EDGES // NW)

    def chunk_body(i, _):
      off = base + i * SEGC
      pltpu.sync_copy(dst_hbm.at[pl.ds(off, SEGC)], dst_v)
      pltpu.sync_copy(msg_hbm.at[pl.ds(off, SEGC), :], msg_v)
      pltpu.sync_copy(msg_v, aggr_s.at[dst_v], add=True)
      return 0

    lax.fori_loop(0, NSEGC, chunk_body, 0)
    plsc.subcore_barrier()

    for z in range(NZ):
      rows = pl.ds(row0 + z * ZC, ZC)
      pltpu.sync_copy(aggr_s.at[rows, :], z_v)
      pltpu.sync_copy(z_v, out_hbm.at[cid, rows, :])

  return sc_seg


_sc_msg_128 = _make_sc_msg(HID)
_sc_seg_128 = _make_sc_seg(HID)


_EB = 1000


def _edge_embed_body(ea_ref, w1_ref, b1_ref, w2_ref, b2_ref, w3_ref, b3_ref,
                     w4_ref, b4_ref, w5_ref, b5_ref,
                     e1_ref, e2_ref, e3_ref, e4_ref, e5_ref):
  ea = ea_ref[...]
  e1_ref[...] = jnp.dot(ea, w1_ref[...], preferred_element_type=jnp.float32) + b1_ref[...]
  e2_ref[...] = jnp.dot(ea, w2_ref[...], preferred_element_type=jnp.float32) + b2_ref[...]
  e3_ref[...] = jnp.dot(ea, w3_ref[...], preferred_element_type=jnp.float32) + b3_ref[...]
  e4_ref[...] = jnp.dot(ea, w4_ref[...], preferred_element_type=jnp.float32) + b4_ref[...]
  e5_ref[...] = jnp.dot(ea, w5_ref[...], preferred_element_type=jnp.float32) + b5_ref[...]


def _edge_embed(edge_attr, ws, bs):
  full = lambda shape: pl.BlockSpec(shape, lambda i: (0, 0))
  in_specs = [pl.BlockSpec((_EB, 4), lambda i: (i, 0))]
  for w, b in zip(ws, bs):
    in_specs.append(full(w.shape))
    in_specs.append(full(b.shape))
  args = [edge_attr]
  for w, b in zip(ws, bs):
    args.extend([w, b])
  out_dims = [w.shape[1] for w in ws]
  return pl.pallas_call(
      _edge_embed_body,
      grid=(N_EDGES // _EB,),
      in_specs=in_specs,
      out_specs=[pl.BlockSpec((_EB, d), lambda i: (i, 0)) for d in out_dims],
      out_shape=[jax.ShapeDtypeStruct((N_EDGES, d), jnp.float32) for d in out_dims],
  )(*args)


def _dense_body(h_ref, p0_ref, p1_ref, w1_ref, b1_ref, w2_ref, b2_ref, z2_ref):
  z = h_ref[...] + p0_ref[0] + p1_ref[0]
  z1 = jnp.maximum(
      jnp.dot(z, w1_ref[...], preferred_element_type=jnp.float32) + b1_ref[...], 0.0)
  z2_ref[...] = jnp.dot(z1, w2_ref[...], preferred_element_type=jnp.float32) + b2_ref[...]


def _dense(h, parts, w1, b1, w2, b2):
  din = h.shape[1]
  full = lambda shape: pl.BlockSpec(shape, lambda i: (0,) * len(shape))
  return pl.pallas_call(
      _dense_body,
      grid=(NB,),
      in_specs=[
          pl.BlockSpec((ROW_BLK, din), lambda i: (i, 0)),
          pl.BlockSpec((1, ROW_BLK, din), lambda i: (0, i, 0)),
          pl.BlockSpec((1, ROW_BLK, din), lambda i: (1, i, 0)),
          full((din, HID)),
          full((1, HID)),
          full((HID, HID)),
          full((1, HID)),
      ],
      out_specs=pl.BlockSpec((ROW_BLK, HID), lambda i: (i, 0)),
      out_shape=jax.ShapeDtypeStruct((N_PAD, HID), jnp.float32),
  )(h, parts, parts, w1, b1, w2, b2)


def _bn_body(z2_ref, mean_ref, var_ref, g_ref, b_ref, h_ref):
  zn = ((z2_ref[...] - mean_ref[...]) / jnp.sqrt(var_ref[...] + 1e-5)
        * g_ref[...] + b_ref[...])
  h_ref[...] = jnp.maximum(zn, 0.0)


def _bn_apply(z2, mean, var, gamma, beta):
  full = lambda shape: pl.BlockSpec(shape, lambda i: (0,) * len(shape))
  return pl.pallas_call(
      _bn_body,
      grid=(NB,),
      in_specs=[
          pl.BlockSpec((ROW_BLK, HID), lambda i: (i, 0)),
          full((1, HID)),
          full((1, HID)),
          full((1, HID)),
          full((1, HID)),
      ],
      out_specs=pl.BlockSpec((ROW_BLK, HID), lambda i: (i, 0)),
      out_shape=jax.ShapeDtypeStruct((N_PAD, HID), jnp.float32),
  )(z2, mean, var, gamma, beta)


def _head_body(f1_ref, f2_ref, f3_ref, f4_ref, f5_ref, w_ref, b1_ref,
               w2_ref, b2_ref, o_ref):
  w = w_ref[...]
  acc = jnp.dot(f1_ref[...], w[0:HID], preferred_element_type=jnp.float32)
  acc += jnp.dot(f2_ref[...], w[HID:2 * HID], preferred_element_type=jnp.float32)
  acc += jnp.dot(f3_ref[...], w[2 * HID:3 * HID], preferred_element_type=jnp.float32)
  acc += jnp.dot(f4_ref[...], w[3 * HID:4 * HID], preferred_element_type=jnp.float32)
  acc += jnp.dot(f5_ref[...], w[4 * HID:5 * HID], preferred_element_type=jnp.float32)
  a = jnp.maximum(acc + b1_ref[...], 0.0)
  o_ref[...] = jnp.dot(a, w2_ref[...], preferred_element_type=jnp.float32) + b2_ref[...]


def _head(feats, fc1w, fc1b, fc2w, fc2b):
  full = lambda shape: pl.BlockSpec(shape, lambda i: (0,) * len(shape))
  return pl.pallas_call(
      _head_body,
      grid=(NB,),
      in_specs=[pl.BlockSpec((ROW_BLK, HID), lambda i: (i, 0))] * 5 + [
          full((5 * HID, HID)),
          full((1, HID)),
          full((HID, OUT_DIM)),
          full((1, OUT_DIM)),
      ],
      out_specs=pl.BlockSpec((ROW_BLK, OUT_DIM), lambda i: (i, 0)),
      out_shape=jax.ShapeDtypeStruct((N_PAD, OUT_DIM), jnp.float32),
  )(*feats, fc1w, fc1b, fc2w, fc2b)


def kernel(x, edge_attr, params, edge_index):
  src = edge_index[0]
  dst = edge_index[1]
  layers = params["layers"]

  h = jnp.pad(x, ((0, N_PAD - N_NODES), (0, D1 - x.shape[1])))


  ws = [jnp.pad(layers[0]["We"], ((0, 0), (0, D1 - layers[0]["We"].shape[1])))]
  bs = [jnp.pad(layers[0]["be"], (0, D1 - layers[0]["be"].shape[0])).reshape(1, D1)]
  for p in layers[1:]:
    ws.append(p["We"])
    bs.append(p["be"].reshape(1, HID))
  e_list = _edge_embed(edge_attr, ws, bs)

  feats = []
  for i, p in enumerate(layers):
    if i == 0:
      w1 = jnp.pad(p["W1"], ((0, D1 - p["W1"].shape[0]), (0, 0)))
    else:
      w1 = p["W1"]
    msg = _sc_msg_128(h, e_list[i], src)
    parts = _sc_seg_128(msg, dst)
    z2 = _dense(h, parts, w1, p["b1"].reshape(1, HID),
                p["W2"], p["b2"].reshape(1, HID))
    zv = z2[:N_NODES]
    mean = jnp.mean(zv, axis=0).reshape(1, HID)
    var = jnp.var(zv, axis=0).reshape(1, HID)
    h = _bn_apply(z2, mean, var, p["gamma"].reshape(1, HID), p["beta"].reshape(1, HID))
    feats.append(h)

  o = _head(feats, params["fc1W"], params["fc1b"].reshape(1, HID),
            params["fc2W"], params["fc2b"].reshape(1, OUT_DIM))
  return o[:N_NODES]
